# baseline (speedup 1.0000x reference)
import functools
import math

import jax
import jax.numpy as jnp
import numpy as np
from jax.experimental import pallas as pl
from jax.experimental.pallas import tpu as pltpu


def _dft_constants(L: int, LFP: int):
    Lf = L // 2 + 1
    f = np.arange(LFP, dtype=np.int64)[:, None]
    t = np.arange(L, dtype=np.int64)[None, :]
    ang = 2.0 * np.pi * ((f * t) % L).astype(np.float64) / L
    out = []
    for m in (np.cos(ang), np.sin(ang)):
        m[Lf:, :] = 0.0
        m32 = m.astype(np.float32)
        hi = m32.astype(jnp.bfloat16)
        lo = (m32 - hi.astype(np.float32)).astype(jnp.bfloat16)
        out.append((jnp.asarray(hi), jnp.asarray(lo)))
    return out[0], out[1]


def _dot3(ah, al, bh, bl, dn):
    kw = dict(dimension_numbers=dn, preferred_element_type=jnp.float32)
    return (jax.lax.dot_general(ah, bh, **kw)
            + jax.lax.dot_general(ah, bl, **kw)
            + jax.lax.dot_general(al, bh, **kw))


def _split_bf16(x):
    hi = x.astype(jnp.bfloat16)
    lo = (x - hi.astype(jnp.float32)).astype(jnp.bfloat16)
    return hi, lo


_FB = 256


def _corr_kernel(L, Lf, q_ref, k_ref, cth_ref, ctl_ref, sth_ref, stl_ref,
                 corr_ref):
    dh = q_ref.shape[-1]
    lfp = cth_ref.shape[0]
    x = jnp.concatenate(
        [q_ref[0], k_ref[0], q_ref[1], k_ref[1]], axis=1)
    xh, xl = _split_bf16(x)
    dn = (((1,), (0,)), ((), ()))
    dn2 = (((0,), (0,)), ((), ()))
    nfb = lfp // _FB

    res_re_l = []
    res_im_l = []
    for i in range(nfb):
        f0 = i * _FB
        cth = cth_ref[f0:f0 + _FB, :]
        ctl = ctl_ref[f0:f0 + _FB, :]
        sth = sth_ref[f0:f0 + _FB, :]
        stl = stl_ref[f0:f0 + _FB, :]
        a = _dot3(cth, ctl, xh, xl, dn)
        b = _dot3(sth, stl, xh, xl, dn)
        fidx = f0 + jax.lax.broadcasted_iota(jnp.int32, (_FB, 1), 0)
        w = jnp.where((fidx == 0) | (fidx == L // 2), 1.0, 2.0) / (L * dh)
        res_re = []
        res_im = []
        for h in range(2):
            aq, ak = a[:, 2 * h * dh:(2 * h + 1) * dh] * w, \
                a[:, (2 * h + 1) * dh:(2 * h + 2) * dh]
            bq, bk = b[:, 2 * h * dh:(2 * h + 1) * dh] * w, \
                b[:, (2 * h + 1) * dh:(2 * h + 2) * dh]
            res_re.append(jnp.sum(aq * ak + bq * bk, axis=1, keepdims=True))
            res_im.append(jnp.sum(aq * bk - bq * ak, axis=1, keepdims=True))
        res_re_l.append(jnp.concatenate(res_re, axis=1))
        res_im_l.append(jnp.concatenate(res_im, axis=1))

    res_re_all = jnp.concatenate(res_re_l, axis=0)
    res_im_all = jnp.concatenate(res_im_l, axis=0)
    reh, rel = _split_bf16(res_re_all)
    imh, iml = _split_bf16(res_im_all)

    acc = jnp.zeros((2, L), jnp.float32)
    for i in range(nfb):
        f0 = i * _FB
        sl = (slice(f0, f0 + _FB), slice(None))
        c_re = _dot3(reh[sl], rel[sl], cth_ref[sl], ctl_ref[sl], dn2)
        c_im = _dot3(imh[sl], iml[sl], sth_ref[sl], stl_ref[sl], dn2)
        acc = acc + c_re - c_im
    corr_ref[0] = acc


def _agg_kernel(K, corr_ref, v_ref, out_ref, v2_ref, g_ref, sems):
    L = v_ref.shape[1]
    cp0 = pltpu.make_async_copy(v_ref.at[0], v2_ref.at[pl.ds(0, L), :],
                                sems.at[K])
    cp1 = pltpu.make_async_copy(v_ref.at[0], v2_ref.at[pl.ds(L, L), :],
                                sems.at[K + 1])
    cp0.start()
    cp1.start()

    r = corr_ref[0]
    iota = jax.lax.broadcasted_iota(jnp.int32, r.shape, 1)
    neg = jnp.float32(-jnp.inf)
    vals = []
    idxs = []
    for _ in range(K):
        m = jnp.max(r)
        i = jnp.min(jnp.where(r == m, iota, L))
        vals.append(m)
        idxs.append(i)
        r = jnp.where(iota == i, neg, r)
    m0 = functools.reduce(jnp.maximum, vals)
    es = [jnp.exp(w - m0) for w in vals]
    s = functools.reduce(lambda x, y: x + y, es)

    cp0.wait()
    cp1.wait()
    cps = []
    for j in range(K):
        cp = pltpu.make_async_copy(v2_ref.at[pl.ds(L - idxs[j], L), :],
                                   g_ref.at[j], sems.at[j])
        cp.start()
        cps.append(cp)
    cps[0].wait()
    acc = (es[0] / s) * g_ref[0]
    for j in range(1, K):
        cps[j].wait()
        acc = acc + (es[j] / s) * g_ref[j]
    out_ref[0] = acc


def kernel(q, k, v):
    B, H, L, Dh = q.shape
    BH = B * H
    Lf = L // 2 + 1
    LFP = ((Lf + _FB - 1) // _FB) * _FB
    K = max(1, int(math.log(L + 1)))

    (cth, ctl), (sth, stl) = _dft_constants(L, LFP)
    q3 = q.reshape(BH, L, Dh)
    k3 = k.reshape(BH, L, Dh)
    v3 = v.reshape(BH, L, Dh)

    corr = pl.pallas_call(
        functools.partial(_corr_kernel, L, Lf),
        grid=(BH // 2,),
        in_specs=[
            pl.BlockSpec((2, L, Dh), lambda i: (i, 0, 0)),
            pl.BlockSpec((2, L, Dh), lambda i: (i, 0, 0)),
            pl.BlockSpec((LFP, L), lambda i: (0, 0)),
            pl.BlockSpec((LFP, L), lambda i: (0, 0)),
            pl.BlockSpec((LFP, L), lambda i: (0, 0)),
            pl.BlockSpec((LFP, L), lambda i: (0, 0)),
        ],
        out_specs=pl.BlockSpec((1, 2, L), lambda i: (i, 0, 0)),
        out_shape=jax.ShapeDtypeStruct((BH // 2, 2, L), jnp.float32),
        compiler_params=pltpu.CompilerParams(
            dimension_semantics=("arbitrary",)),
    )(q3, k3, cth, ctl, sth, stl)
    corr = corr.reshape(BH, 1, L)

    out = pl.pallas_call(
        functools.partial(_agg_kernel, K),
        grid=(BH,),
        in_specs=[
            pl.BlockSpec((1, 1, L), lambda i: (i, 0, 0)),
            pl.BlockSpec((1, L, Dh), lambda i: (i, 0, 0)),
        ],
        out_specs=pl.BlockSpec((1, L, Dh), lambda i: (i, 0, 0)),
        out_shape=jax.ShapeDtypeStruct((BH, L, Dh), jnp.float32),
        scratch_shapes=[
            pltpu.VMEM((2 * L, Dh), jnp.float32),
            pltpu.VMEM((K, L, Dh), jnp.float32),
            pltpu.SemaphoreType.DMA((K + 2,)),
        ],
        compiler_params=pltpu.CompilerParams(
            dimension_semantics=("arbitrary",)),
    )(corr, v3)

    return out.reshape(B, H, L, Dh)
